# baseline (speedup 1.0000x reference)
import functools

import jax
import jax.numpy as jnp
from jax import lax
from jax.experimental import pallas as pl
from jax.experimental.pallas import tpu as pltpu
from jax.experimental.pallas import tpu_sc as plsc

N = 10000
NP = 10240
E = 320000
D = 128
H = 16
CLS = 5
NC, NS, L = 2, 16, 16
NW = NC * NS
EB = 128
NB = 80
EPAD = NW * NB * EB
RPS = NP // NS
NBUF = 8

_mesh = plsc.VectorSubcoreMesh(
    core_axis_name="c", subcore_axis_name="s", num_cores=NC, num_subcores=NS)


@functools.partial(
    pl.kernel,
    out_type=jax.ShapeDtypeStruct((NW, NP), jnp.float32),
    mesh=_mesh,
    scratch_types=[
        pltpu.VMEM((NB * EB,), jnp.int32),
        pltpu.VMEM((NB * EB,), jnp.float32),
        pltpu.VMEM((NP,), jnp.float32),
    ],
    compiler_params=pltpu.CompilerParams(needs_layout_passes=False),
)
def _deg_sc(col_hbm, ew_hbm, out_hbm, col_v, ew_v, pdeg):
    cid = lax.axis_index("c")
    sid = lax.axis_index("s")
    wid = sid * NC + cid

    def zero_body(i, carry):
        pdeg[pl.ds(i * L, L)] = jnp.zeros((L,), jnp.float32)
        return carry

    lax.fori_loop(0, NP // L, zero_body, 0)

    pltpu.sync_copy(col_hbm.at[wid], col_v)
    pltpu.sync_copy(ew_hbm.at[wid], ew_v)

    def grp_body(g, carry):
        idx16 = col_v[pl.ds(g * L, L)]
        ew16 = ew_v[pl.ds(g * L, L)]
        plsc.addupdate_scatter(pdeg, [idx16], ew16)
        return carry

    lax.fori_loop(0, NB * EB // L, grp_body, 0)

    pltpu.sync_copy(pdeg, out_hbm.at[wid])


@functools.partial(
    pl.kernel,
    out_type=jax.ShapeDtypeStruct((NC, NP, H), jnp.float32),
    mesh=_mesh,
    scratch_types=[
        pltpu.VMEM((NB, EB), jnp.int32),
        pltpu.VMEM((NB, EB), jnp.int32),
        pltpu.VMEM((NB, EB), jnp.float32),
        [pltpu.VMEM((EB, H), jnp.float32) for _ in range(NBUF)],
        pltpu.VMEM((RPS, H), jnp.float32),
        pltpu.VMEM_SHARED((NP, H), jnp.float32),
        pltpu.VMEM_SHARED((NP, H), jnp.float32),
        [pltpu.SemaphoreType.DMA for _ in range(NBUF)],
        [pltpu.SemaphoreType.DMA for _ in range(NBUF)],
    ],
    compiler_params=pltpu.CompilerParams(use_tc_tiling_on_sc=False),
)
def _prop_sc(g_hbm, row_hbm, col_hbm, ew_hbm, out_hbm,
             row_v, col_v, ew_v, msg, zb_v, acc, gsp, gsem, ssem):
    cid = lax.axis_index("c")
    sid = lax.axis_index("s")
    wid = sid * NC + cid

    def zbuf_body(i, carry):
        zb_v[i, :] = jnp.zeros((H,), jnp.float32)
        return carry

    lax.fori_loop(0, RPS, zbuf_body, 0)
    pltpu.sync_copy(zb_v, acc.at[pl.ds(sid * RPS, RPS)])
    pltpu.sync_copy(g_hbm.at[pl.ds(sid * RPS, RPS)],
                    gsp.at[pl.ds(sid * RPS, RPS)])
    plsc.subcore_barrier()

    pltpu.sync_copy(row_hbm.at[wid], row_v)
    pltpu.sync_copy(col_hbm.at[wid], col_v)
    pltpu.sync_copy(ew_hbm.at[wid], ew_v)

    def _scale(k, b):
        def scale_body(g, c2):
            ew16 = ew_v[b, pl.ds(g * L, L)]
            base = g * L
            for j in range(L):
                msg[k][base + j, :] = msg[k][base + j, :] * ew16[j]
            return c2

        lax.fori_loop(0, EB // L, scale_body, 0)

    _run_blocks(gsp, row_v, col_v, ew_v, msg, acc, gsem, ssem, _scale)

    plsc.subcore_barrier()
    pltpu.sync_copy(acc.at[pl.ds(sid * RPS, RPS)],
                    out_hbm.at[cid].at[pl.ds(sid * RPS, RPS)])


def _run_blocks(g_hbm, row_v, col_v, ew_v, msg, acc, gsem, ssem, _scale):
    for k in range(4):
        pltpu.async_copy(g_hbm.at[row_v.at[k]], msg[k], gsem[k])

    def steady_body(p, carry):
        for k in range(NBUF):
            b = p * NBUF + k
            kf = (k + 4) % NBUF

            @pl.when(jnp.logical_or(p >= 1, k >= 4))
            def _drain():
                pltpu.make_async_copy(
                    msg[kf], acc.at[col_v.at[b - 4]], ssem[kf]).wait()

            @pl.when(jnp.logical_or(p <= NB // NBUF - 2, k < 4))
            def _fire():
                pltpu.async_copy(g_hbm.at[row_v.at[b + 4]], msg[kf], gsem[kf])

            pltpu.make_async_copy(g_hbm.at[row_v.at[b]], msg[k],
                                  gsem[k]).wait()
            _scale(k, b)
            pltpu.async_copy(msg[k], acc.at[col_v.at[b]], ssem[k], add=True)
        return carry

    lax.fori_loop(0, NB // NBUF, steady_body, 0)
    for i in range(4):
        k = (4 + i) % NBUF
        pltpu.make_async_copy(msg[k], acc.at[col_v.at[NB - 4 + i]],
                              ssem[k]).wait()


def _tc_a0_body(x_ref, w1_ref, h1_ref):
    h1_ref[...] = jnp.dot(x_ref[...], w1_ref[...],
                          preferred_element_type=jnp.float32,
                          precision=lax.Precision.HIGHEST)


_tc_a0 = pl.pallas_call(
    _tc_a0_body,
    out_shape=jax.ShapeDtypeStruct((N, H), jnp.float32),
)


def _tc_a1_body(degp_ref, h1_ref, g1_ref, dinv_ref):
    deg = jnp.sum(degp_ref[:, :N], axis=0) + 1.0
    dinv = lax.rsqrt(deg)
    g1_ref[:N, :] = h1_ref[...] * dinv[:, None]
    g1_ref[pl.ds(N, NP - N), :] = jnp.zeros((NP - N, H), jnp.float32)
    dinv_ref[...] = dinv


_tc_a1 = pl.pallas_call(
    _tc_a1_body,
    out_shape=[
        jax.ShapeDtypeStruct((NP, H), jnp.float32),
        jax.ShapeDtypeStruct((N,), jnp.float32),
    ],
)


def _tc_b_body(sp_ref, g1_ref, dinv_ref, w2_ref, b1_ref, g2_ref):
    dinv = dinv_ref[...]
    s = sp_ref[0, :N, :] + sp_ref[1, :N, :]
    z = (s + g1_ref[:N, :]) * dinv[:, None]
    z = z + b1_ref[...][None, :]
    a = jnp.maximum(z, 0.0)
    h2 = jnp.dot(a, w2_ref[...],
                 preferred_element_type=jnp.float32,
                 precision=lax.Precision.HIGHEST)
    g2_ref[:N, :] = h2 * dinv[:, None]
    g2_ref[pl.ds(N, NP - N), :] = jnp.zeros((NP - N, H), jnp.float32)


_tc_b = pl.pallas_call(
    _tc_b_body,
    out_shape=jax.ShapeDtypeStruct((NP, H), jnp.float32),
)


def _tc_c_body(sp_ref, g2_ref, dinv_ref, b2_ref, out_ref):
    dinv = dinv_ref[...]
    z = (sp_ref[0, :N, :] + sp_ref[1, :N, :] + g2_ref[:N, :]) * dinv[:, None]
    logits = z[:, :CLS] + b2_ref[...][None, :]
    m = jnp.max(logits, axis=1, keepdims=True)
    lse = jnp.log(jnp.sum(jnp.exp(logits - m), axis=1, keepdims=True)) + m
    out_ref[...] = logits - lse


_tc_c = pl.pallas_call(
    _tc_c_body,
    out_shape=jax.ShapeDtypeStruct((N, CLS), jnp.float32),
)


def kernel(x, edge_index, edge_weight, W1, b1, W2, b2):
    pad = EPAD - E
    row = jnp.concatenate(
        [edge_index[0].astype(jnp.int32), jnp.zeros((pad,), jnp.int32)])
    col = jnp.concatenate(
        [edge_index[1].astype(jnp.int32), jnp.zeros((pad,), jnp.int32)])
    ew = jnp.concatenate(
        [edge_weight, jnp.zeros((pad,), edge_weight.dtype)])
    row = row.reshape(NW, NB, EB)
    col = col.reshape(NW, NB, EB)
    ew = ew.reshape(NW, NB, EB)
    degp = _deg_sc(col.reshape(NW, NB * EB),
                   ew.reshape(NW, NB * EB))
    h1 = _tc_a0(x, W1)
    g1, dinv = _tc_a1(degp, h1)
    s1 = _prop_sc(g1, row, col, ew)
    w2p = jnp.zeros((H, H), W2.dtype).at[:, :CLS].set(W2)
    g2 = _tc_b(s1, g1, dinv, w2p, b1)
    s2 = _prop_sc(g2, row, col, ew)
    return _tc_c(s2, g2, dinv, b2)


# speedup vs baseline: 1.0483x; 1.0483x over previous
import functools

import jax
import jax.numpy as jnp
from jax import lax
from jax.experimental import pallas as pl
from jax.experimental.pallas import tpu as pltpu
from jax.experimental.pallas import tpu_sc as plsc

N = 10000
NP = 10240
E = 320000
D = 128
H = 16
CLS = 5
NC, NS, L = 2, 16, 16
NW = NC * NS
EB = 128
NB = 80
EPAD = NW * NB * EB
RPS = NP // NS
NBUF = 8

_mesh = plsc.VectorSubcoreMesh(
    core_axis_name="c", subcore_axis_name="s", num_cores=NC, num_subcores=NS)


@functools.partial(
    pl.kernel,
    out_type=jax.ShapeDtypeStruct((NW, NP), jnp.float32),
    mesh=_mesh,
    scratch_types=[
        pltpu.VMEM((NB, EB), jnp.int32),
        pltpu.VMEM((NB, EB), jnp.float32),
        pltpu.VMEM((RPS,), jnp.float32),
        pltpu.VMEM_SHARED((NP,), jnp.float32),
        pltpu.SemaphoreType.DMA,
    ],
)
def _deg_sc(col_hbm, ew_hbm, out_hbm, col_v, ew_v, zb_v, acc, sem):
    cid = lax.axis_index("c")
    sid = lax.axis_index("s")
    wid = sid * NC + cid

    def zero_body(i, carry):
        zb_v[pl.ds(i * L, L)] = jnp.zeros((L,), jnp.float32)
        return carry

    lax.fori_loop(0, RPS // L, zero_body, 0)
    pltpu.sync_copy(zb_v, acc.at[pl.ds(sid * RPS, RPS)])
    plsc.subcore_barrier()

    pltpu.sync_copy(col_hbm.at[wid], col_v)
    pltpu.sync_copy(ew_hbm.at[wid], ew_v)

    def blk_body(b, carry):
        pltpu.async_copy(ew_v.at[b], acc.at[col_v.at[b]], sem, add=True)
        return carry

    lax.fori_loop(0, NB, blk_body, 0)

    def drain_body(b, carry):
        pltpu.make_async_copy(ew_v.at[b], acc.at[col_v.at[b]], sem).wait()
        return carry

    lax.fori_loop(0, NB, drain_body, 0)

    plsc.subcore_barrier()
    pltpu.sync_copy(acc.at[pl.ds(sid * RPS, RPS)],
                    out_hbm.at[cid].at[pl.ds(sid * RPS, RPS)])


@functools.partial(
    pl.kernel,
    out_type=jax.ShapeDtypeStruct((NC, NP, H), jnp.float32),
    mesh=_mesh,
    scratch_types=[
        pltpu.VMEM((NB, EB), jnp.int32),
        pltpu.VMEM((NB, EB), jnp.int32),
        pltpu.VMEM((NB, EB), jnp.float32),
        [pltpu.VMEM((EB, H), jnp.float32) for _ in range(NBUF)],
        pltpu.VMEM((64, H), jnp.float32),
        pltpu.VMEM_SHARED((NP, H), jnp.float32),
        pltpu.VMEM_SHARED((NP, H), jnp.float32),
        [pltpu.SemaphoreType.DMA for _ in range(NBUF)],
        [pltpu.SemaphoreType.DMA for _ in range(NBUF)],
    ],
    compiler_params=pltpu.CompilerParams(use_tc_tiling_on_sc=False),
)
def _prop_sc(g_hbm, row_hbm, col_hbm, ew_hbm, out_hbm,
             row_v, col_v, ew_v, msg, zb_v, acc, gsp, gsem, ssem):
    cid = lax.axis_index("c")
    sid = lax.axis_index("s")
    wid = sid * NC + cid

    def zbuf_body(i, carry):
        zb_v[i, :] = jnp.zeros((H,), jnp.float32)
        return carry

    lax.fori_loop(0, 64, zbuf_body, 0)
    for zk in range(RPS // 64):
        pltpu.sync_copy(zb_v, acc.at[pl.ds(sid * RPS + zk * 64, 64)])
    pltpu.sync_copy(g_hbm.at[pl.ds(sid * RPS, RPS)],
                    gsp.at[pl.ds(sid * RPS, RPS)])
    plsc.subcore_barrier()

    pltpu.sync_copy(row_hbm.at[wid], row_v)
    pltpu.sync_copy(col_hbm.at[wid], col_v)
    pltpu.sync_copy(ew_hbm.at[wid], ew_v)

    def _scale(k, b):
        def scale_body(g, c2):
            ew16 = ew_v[b, pl.ds(g * L, L)]
            base = g * L
            for j in range(L):
                msg[k][base + j, :] = msg[k][base + j, :] * ew16[j]
            return c2

        lax.fori_loop(0, EB // L, scale_body, 0)

    _run_blocks(gsp, row_v, col_v, ew_v, msg, acc, gsem, ssem, _scale)

    plsc.subcore_barrier()
    pltpu.sync_copy(acc.at[pl.ds(sid * RPS, RPS)],
                    out_hbm.at[cid].at[pl.ds(sid * RPS, RPS)])


def _run_blocks(g_hbm, row_v, col_v, ew_v, msg, acc, gsem, ssem, _scale):
    for k in range(4):
        pltpu.async_copy(g_hbm.at[row_v.at[k]], msg[k], gsem[k])

    def steady_body(p, carry):
        for k in range(NBUF):
            b = p * NBUF + k
            kf = (k + 4) % NBUF

            @pl.when(jnp.logical_or(p >= 1, k >= 4))
            def _drain():
                pltpu.make_async_copy(
                    msg[kf], acc.at[col_v.at[b - 4]], ssem[kf]).wait()

            @pl.when(jnp.logical_or(p <= NB // NBUF - 2, k < 4))
            def _fire():
                pltpu.async_copy(g_hbm.at[row_v.at[b + 4]], msg[kf], gsem[kf])

            pltpu.make_async_copy(g_hbm.at[row_v.at[b]], msg[k],
                                  gsem[k]).wait()
            _scale(k, b)
            pltpu.async_copy(msg[k], acc.at[col_v.at[b]], ssem[k], add=True)
        return carry

    lax.fori_loop(0, NB // NBUF, steady_body, 0)
    for i in range(4):
        k = (4 + i) % NBUF
        pltpu.make_async_copy(msg[k], acc.at[col_v.at[NB - 4 + i]],
                              ssem[k]).wait()


def _tc_a0_body(x_ref, w1_ref, h1_ref):
    h1_ref[...] = jnp.dot(x_ref[...], w1_ref[...],
                          preferred_element_type=jnp.float32,
                          precision=lax.Precision.HIGHEST)


_tc_a0 = pl.pallas_call(
    _tc_a0_body,
    out_shape=jax.ShapeDtypeStruct((N, H), jnp.float32),
)


def _tc_a1_body(degp_ref, h1_ref, g1_ref, dinv_ref):
    deg = degp_ref[0, :N] + degp_ref[1, :N] + 1.0
    dinv = lax.rsqrt(deg)
    g1_ref[:N, :] = h1_ref[...] * dinv[:, None]
    g1_ref[pl.ds(N, NP - N), :] = jnp.zeros((NP - N, H), jnp.float32)
    dinv_ref[...] = dinv


_tc_a1 = pl.pallas_call(
    _tc_a1_body,
    out_shape=[
        jax.ShapeDtypeStruct((NP, H), jnp.float32),
        jax.ShapeDtypeStruct((N,), jnp.float32),
    ],
)


def _tc_b_body(sp_ref, g1_ref, dinv_ref, w2_ref, b1_ref, g2_ref):
    dinv = dinv_ref[...]
    s = sp_ref[0, :N, :] + sp_ref[1, :N, :]
    z = (s + g1_ref[:N, :]) * dinv[:, None]
    z = z + b1_ref[...][None, :]
    a = jnp.maximum(z, 0.0)
    h2 = jnp.dot(a, w2_ref[...],
                 preferred_element_type=jnp.float32,
                 precision=lax.Precision.HIGHEST)
    g2_ref[:N, :] = h2 * dinv[:, None]
    g2_ref[pl.ds(N, NP - N), :] = jnp.zeros((NP - N, H), jnp.float32)


_tc_b = pl.pallas_call(
    _tc_b_body,
    out_shape=jax.ShapeDtypeStruct((NP, H), jnp.float32),
)


def _tc_c_body(sp_ref, g2_ref, dinv_ref, b2_ref, out_ref):
    dinv = dinv_ref[...]
    z = (sp_ref[0, :N, :] + sp_ref[1, :N, :] + g2_ref[:N, :]) * dinv[:, None]
    logits = z[:, :CLS] + b2_ref[...][None, :]
    m = jnp.max(logits, axis=1, keepdims=True)
    lse = jnp.log(jnp.sum(jnp.exp(logits - m), axis=1, keepdims=True)) + m
    out_ref[...] = logits - lse


_tc_c = pl.pallas_call(
    _tc_c_body,
    out_shape=jax.ShapeDtypeStruct((N, CLS), jnp.float32),
)


def kernel(x, edge_index, edge_weight, W1, b1, W2, b2):
    pad = EPAD - E
    row = jnp.concatenate(
        [edge_index[0].astype(jnp.int32), jnp.zeros((pad,), jnp.int32)])
    col = jnp.concatenate(
        [edge_index[1].astype(jnp.int32), jnp.zeros((pad,), jnp.int32)])
    ew = jnp.concatenate(
        [edge_weight, jnp.zeros((pad,), edge_weight.dtype)])
    row = row.reshape(NW, NB, EB)
    col = col.reshape(NW, NB, EB)
    ew = ew.reshape(NW, NB, EB)
    degp = _deg_sc(col, ew)
    h1 = _tc_a0(x, W1)
    g1, dinv = _tc_a1(degp, h1)
    s1 = _prop_sc(g1, row, col, ew)
    w2p = jnp.zeros((H, H), W2.dtype).at[:, :CLS].set(W2)
    g2 = _tc_b(s1, g1, dinv, w2p, b1)
    s2 = _prop_sc(g2, row, col, ew)
    return _tc_c(s2, g2, dinv, b2)


# speedup vs baseline: 1.0521x; 1.0036x over previous
import functools

import jax
import jax.numpy as jnp
from jax import lax
from jax.experimental import pallas as pl
from jax.experimental.pallas import tpu as pltpu
from jax.experimental.pallas import tpu_sc as plsc

N = 10000
NP = 10240
E = 320000
D = 128
H = 16
CLS = 5
NC, NS, L = 2, 16, 16
NW = NC * NS
EB = 128
NB = 80
EPAD = NW * NB * EB
RPS = NP // NS
NBUF = 8

_mesh = plsc.VectorSubcoreMesh(
    core_axis_name="c", subcore_axis_name="s", num_cores=NC, num_subcores=NS)


@functools.partial(
    pl.kernel,
    out_type=jax.ShapeDtypeStruct((NC, NP), jnp.float32),
    mesh=_mesh,
    scratch_types=[
        pltpu.VMEM((NB, EB), jnp.int32),
        pltpu.VMEM((NB, EB), jnp.float32),
        pltpu.VMEM((RPS,), jnp.float32),
        pltpu.VMEM_SHARED((NP,), jnp.float32),
        pltpu.SemaphoreType.DMA,
    ],
)
def _deg_sc(col_hbm, ew_hbm, out_hbm, col_v, ew_v, zb_v, acc, sem):
    cid = lax.axis_index("c")
    sid = lax.axis_index("s")
    wid = sid * NC + cid

    def zero_body(i, carry):
        zb_v[pl.ds(i * L, L)] = jnp.zeros((L,), jnp.float32)
        return carry

    lax.fori_loop(0, RPS // L, zero_body, 0)
    pltpu.sync_copy(zb_v, acc.at[pl.ds(sid * RPS, RPS)])
    plsc.subcore_barrier()

    pltpu.sync_copy(col_hbm.at[wid], col_v)
    pltpu.sync_copy(ew_hbm.at[wid], ew_v)

    def blk_body(b, carry):
        pltpu.async_copy(ew_v.at[b], acc.at[col_v.at[b]], sem, add=True)
        return carry

    lax.fori_loop(0, NB, blk_body, 0)

    def drain_body(b, carry):
        pltpu.make_async_copy(ew_v.at[b], acc.at[col_v.at[b]], sem).wait()
        return carry

    lax.fori_loop(0, NB, drain_body, 0)

    plsc.subcore_barrier()
    pltpu.sync_copy(acc.at[pl.ds(sid * RPS, RPS)],
                    out_hbm.at[cid].at[pl.ds(sid * RPS, RPS)])


@functools.partial(
    pl.kernel,
    out_type=jax.ShapeDtypeStruct((NC, NP, H), jnp.float32),
    mesh=_mesh,
    scratch_types=[
        pltpu.VMEM((NB, EB), jnp.int32),
        pltpu.VMEM((NB, EB), jnp.int32),
        pltpu.VMEM((NB, EB), jnp.float32),
        [pltpu.VMEM((EB, H), jnp.float32) for _ in range(NBUF)],
        pltpu.VMEM((64, H), jnp.float32),
        pltpu.VMEM_SHARED((NP, H), jnp.float32),
        pltpu.VMEM_SHARED((NP, H), jnp.float32),
        [pltpu.SemaphoreType.DMA for _ in range(NBUF)],
        [pltpu.SemaphoreType.DMA for _ in range(NBUF)],
    ],
    compiler_params=pltpu.CompilerParams(use_tc_tiling_on_sc=False),
)
def _prop_sc(g_hbm, row_hbm, col_hbm, ew_hbm, out_hbm,
             row_v, col_v, ew_v, msg, zb_v, acc, gsp, gsem, ssem):
    cid = lax.axis_index("c")
    sid = lax.axis_index("s")
    wid = sid * NC + cid

    def zbuf_body(i, carry):
        zb_v[i, :] = jnp.zeros((H,), jnp.float32)
        return carry

    lax.fori_loop(0, 64, zbuf_body, 0)
    for zk in range(RPS // 64):
        pltpu.sync_copy(zb_v, acc.at[pl.ds(sid * RPS + zk * 64, 64)])
    pltpu.sync_copy(g_hbm.at[pl.ds(sid * RPS, RPS)],
                    gsp.at[pl.ds(sid * RPS, RPS)])
    plsc.subcore_barrier()

    pltpu.sync_copy(row_hbm.at[wid], row_v)
    pltpu.sync_copy(col_hbm.at[wid], col_v)
    pltpu.sync_copy(ew_hbm.at[wid], ew_v)

    def _scale(k, b):
        def scale_body(g, c2):
            ew16 = ew_v[b, pl.ds(g * L, L)]
            base = g * L
            for j in range(L):
                msg[k][base + j, :] = msg[k][base + j, :] * ew16[j]
            return c2

        lax.fori_loop(0, EB // L, scale_body, 0)

    _run_blocks(gsp, row_v, col_v, ew_v, msg, acc, gsem, ssem, _scale)

    plsc.subcore_barrier()
    pltpu.sync_copy(acc.at[pl.ds(sid * RPS, RPS)],
                    out_hbm.at[cid].at[pl.ds(sid * RPS, RPS)])


def _run_blocks(g_hbm, row_v, col_v, ew_v, msg, acc, gsem, ssem, _scale):
    for k in range(4):
        pltpu.async_copy(g_hbm.at[row_v.at[k]], msg[k], gsem[k])

    def steady_body(p, carry):
        for k in range(NBUF):
            b = p * NBUF + k
            kf = (k + 4) % NBUF

            @pl.when(jnp.logical_or(p >= 1, k >= 4))
            def _drain():
                pltpu.make_async_copy(
                    msg[kf], acc.at[col_v.at[b - 4]], ssem[kf]).wait()

            @pl.when(jnp.logical_or(p <= NB // NBUF - 2, k < 4))
            def _fire():
                pltpu.async_copy(g_hbm.at[row_v.at[b + 4]], msg[kf], gsem[kf])

            pltpu.make_async_copy(g_hbm.at[row_v.at[b]], msg[k],
                                  gsem[k]).wait()
            _scale(k, b)
            pltpu.async_copy(msg[k], acc.at[col_v.at[b]], ssem[k], add=True)
        return carry

    lax.fori_loop(0, NB // NBUF, steady_body, 0)
    for i in range(4):
        k = (4 + i) % NBUF
        pltpu.make_async_copy(msg[k], acc.at[col_v.at[NB - 4 + i]],
                              ssem[k]).wait()


def _tc_a0_body(x_ref, w1_ref, h1_ref):
    h1_ref[...] = jnp.dot(x_ref[...], w1_ref[...],
                          preferred_element_type=jnp.float32,
                          precision=lax.Precision.HIGHEST)


_tc_a0 = pl.pallas_call(
    _tc_a0_body,
    out_shape=jax.ShapeDtypeStruct((N, H), jnp.float32),
)


def _tc_a1_body(degp_ref, h1_ref, g1_ref, dinv_ref):
    deg = degp_ref[0, :N] + degp_ref[1, :N] + 1.0
    dinv = lax.rsqrt(deg)
    g1_ref[:N, :] = h1_ref[...] * dinv[:, None]
    g1_ref[pl.ds(N, NP - N), :] = jnp.zeros((NP - N, H), jnp.float32)
    dinv_ref[...] = dinv


_tc_a1 = pl.pallas_call(
    _tc_a1_body,
    out_shape=[
        jax.ShapeDtypeStruct((NP, H), jnp.float32),
        jax.ShapeDtypeStruct((N,), jnp.float32),
    ],
)


def _tc_b_body(sp_ref, g1_ref, dinv_ref, w2_ref, b1_ref, g2_ref):
    dinv = dinv_ref[...]
    s = sp_ref[0, :N, :] + sp_ref[1, :N, :]
    z = (s + g1_ref[:N, :]) * dinv[:, None]
    z = z + b1_ref[...][None, :]
    a = jnp.maximum(z, 0.0)
    h2 = jnp.dot(a, w2_ref[...],
                 preferred_element_type=jnp.float32,
                 precision=lax.Precision.HIGHEST)
    g2_ref[:N, :] = h2 * dinv[:, None]
    g2_ref[pl.ds(N, NP - N), :] = jnp.zeros((NP - N, H), jnp.float32)


_tc_b = pl.pallas_call(
    _tc_b_body,
    out_shape=jax.ShapeDtypeStruct((NP, H), jnp.float32),
)


def _tc_c_body(sp_ref, g2_ref, dinv_ref, b2_ref, out_ref):
    dinv = dinv_ref[...]
    z = (sp_ref[0, :N, :] + sp_ref[1, :N, :] + g2_ref[:N, :]) * dinv[:, None]
    logits = z[:, :CLS] + b2_ref[...][None, :]
    m = jnp.max(logits, axis=1, keepdims=True)
    lse = jnp.log(jnp.sum(jnp.exp(logits - m), axis=1, keepdims=True)) + m
    out_ref[...] = logits - lse


_tc_c = pl.pallas_call(
    _tc_c_body,
    out_shape=jax.ShapeDtypeStruct((N, CLS), jnp.float32),
)


def kernel(x, edge_index, edge_weight, W1, b1, W2, b2):
    pad = EPAD - E
    row = jnp.concatenate(
        [edge_index[0].astype(jnp.int32), jnp.zeros((pad,), jnp.int32)])
    col = jnp.concatenate(
        [edge_index[1].astype(jnp.int32), jnp.zeros((pad,), jnp.int32)])
    ew = jnp.concatenate(
        [edge_weight, jnp.zeros((pad,), edge_weight.dtype)])
    row = row.reshape(NW, NB, EB)
    col = col.reshape(NW, NB, EB)
    ew = ew.reshape(NW, NB, EB)
    degp = _deg_sc(col, ew)
    h1 = _tc_a0(x, W1)
    g1, dinv = _tc_a1(degp, h1)
    s1 = _prop_sc(g1, row, col, ew)
    w2p = jnp.zeros((H, H), W2.dtype).at[:, :CLS].set(W2)
    g2 = _tc_b(s1, g1, dinv, w2p, b1)
    s2 = _prop_sc(g2, row, col, ew)
    return _tc_c(s2, g2, dinv, b2)
